# baseline (speedup 1.0000x reference)
import jax
import jax.numpy as jnp
from jax import lax
from jax.experimental import pallas as pl
from jax.experimental.pallas import tpu as pltpu
from jax.experimental.pallas import tpu_sc as plsc

_NUM_CORES = 2
_NUM_SUBCORES = 16
_NW = _NUM_CORES * _NUM_SUBCORES


def _gather_body(tok_hbm, idx_hbm, out_hbm, idx_v, rows_v, sem):
    b_per_w = idx_v.shape[0]
    wid = lax.axis_index("s") * _NUM_CORES + lax.axis_index("c")
    base = wid * b_per_w
    pltpu.sync_copy(idx_hbm.at[pl.ds(base, b_per_w)], idx_v)
    pltpu.async_copy(tok_hbm.at[idx_v], rows_v, sem).wait()
    pltpu.sync_copy(rows_v, out_hbm.at[pl.ds(base, b_per_w)])


def _sc_gather(tok_emb, idx):
    seq = idx.shape[0]
    emb = tok_emb.shape[1]
    b_per_w = seq // _NW
    mesh = plsc.VectorSubcoreMesh(core_axis_name="c", subcore_axis_name="s")
    return pl.kernel(
        _gather_body,
        mesh=mesh,
        out_type=jax.ShapeDtypeStruct((seq, emb), jnp.float32),
        scratch_types=[
            pltpu.VMEM((b_per_w,), jnp.int32),
            pltpu.VMEM((b_per_w, emb), jnp.float32),
            pltpu.SemaphoreType.DMA,
        ],
    )(tok_emb, idx)


def _matmul_body(x_ref, pos_ref, w_ref, out_ref):
    x = x_ref[...] + pos_ref[...]
    out_ref[...] = lax.dot_general(
        x, w_ref[...], (((1,), (1,)), ((), ())),
        preferred_element_type=jnp.float32)


def _tc_matmul(x, pos_emb, w_out, tile_v=1024):
    seq, emb = x.shape
    vocab = w_out.shape[0]
    grid = (pl.cdiv(vocab, tile_v),)
    return pl.pallas_call(
        _matmul_body,
        grid=grid,
        in_specs=[
            pl.BlockSpec((seq, emb), lambda i: (0, 0)),
            pl.BlockSpec((seq, emb), lambda i: (0, 0)),
            pl.BlockSpec((tile_v, emb), lambda i: (i, 0)),
        ],
        out_specs=pl.BlockSpec((None, seq, tile_v), lambda i: (0, 0, i)),
        out_shape=jax.ShapeDtypeStruct((1, seq, vocab), jnp.float32),
    )(x, pos_emb, w_out)


def kernel(in_idx, tok_emb, pos_emb, W_out):
    batch, seq = in_idx.shape
    vocab, emb = W_out.shape
    idx = in_idx.reshape(seq).astype(jnp.int32)
    x = _sc_gather(tok_emb, idx)
    return _tc_matmul(x, pos_emb, W_out)


# speedup vs baseline: 2.3917x; 2.3917x over previous
import jax
import jax.numpy as jnp
from jax import lax
from jax.experimental import pallas as pl
from jax.experimental.pallas import tpu as pltpu
from jax.experimental.pallas import tpu_sc as plsc

_NUM_CORES = 2
_NUM_SUBCORES = 16
_NW = _NUM_CORES * _NUM_SUBCORES


def _gather_body(tok_hbm, idx_hbm, out_hbm, idx_v, rows_v, sem):
    b_per_w = idx_v.shape[0]
    wid = lax.axis_index("s") * _NUM_CORES + lax.axis_index("c")
    base = wid * b_per_w
    pltpu.sync_copy(idx_hbm.at[pl.ds(base, b_per_w)], idx_v)
    pltpu.async_copy(tok_hbm.at[idx_v], rows_v, sem).wait()
    pltpu.sync_copy(rows_v, out_hbm.at[pl.ds(base, b_per_w)])


def _sc_gather(tok_emb, idx):
    seq = idx.shape[0]
    emb = tok_emb.shape[1]
    b_per_w = seq // _NW
    mesh = plsc.VectorSubcoreMesh(core_axis_name="c", subcore_axis_name="s")
    return pl.kernel(
        _gather_body,
        mesh=mesh,
        out_type=jax.ShapeDtypeStruct((seq, emb), jnp.float32),
        scratch_types=[
            pltpu.VMEM((b_per_w,), jnp.int32),
            pltpu.VMEM((b_per_w, emb), jnp.float32),
            pltpu.SemaphoreType.DMA,
        ],
    )(tok_emb, idx)


def _matmul_body(x_ref, pos_ref, w_ref, out_ref):
    x = x_ref[...] + pos_ref[...]
    out_ref[...] = lax.dot_general(
        x, w_ref[...], (((1,), (1,)), ((), ())),
        preferred_element_type=jnp.float32)


def _tc_matmul(x, pos_emb, w_out, tile_v=2048):
    seq, emb = x.shape
    vocab = w_out.shape[0]
    grid = (pl.cdiv(vocab, tile_v),)
    return pl.pallas_call(
        _matmul_body,
        grid=grid,
        in_specs=[
            pl.BlockSpec((seq, emb), lambda v: (0, 0)),
            pl.BlockSpec((seq, emb), lambda v: (0, 0)),
            pl.BlockSpec((tile_v, emb), lambda v: (v, 0)),
        ],
        out_specs=pl.BlockSpec((seq, tile_v), lambda v: (0, v)),
        out_shape=jax.ShapeDtypeStruct((seq, vocab), jnp.float32),
    )(x, pos_emb, w_out)


def kernel(in_idx, tok_emb, pos_emb, W_out):
    batch, seq = in_idx.shape
    vocab, emb = W_out.shape
    idx = in_idx.reshape(seq).astype(jnp.int32)
    x = _sc_gather(tok_emb, idx)
    logits = _tc_matmul(x, pos_emb, W_out)
    return logits.reshape(batch, seq, vocab)
